# Initial kernel scaffold; baseline (speedup 1.0000x reference)
import jax
import jax.numpy as jnp
from jax.experimental import pallas as pl


def kernel(boxes, scores, labels):
    raise NotImplementedError("write your pallas kernel here")


# speedup vs baseline: 11.3780x; 11.3780x over previous
import jax
import jax.numpy as jnp
from jax.experimental import pallas as pl
from jax.experimental.pallas import tpu as pltpu

SCORE_THRESH = 0.05
NMS_THRESH = 0.5
DETECTIONS_PER_IMG = 300
TOPK_CANDIDATES = 1000
NEG = -1e10
CLASS_OFFSET = 100000.0

K = TOPK_CANDIDATES
KP = 1024
M = DETECTIONS_PER_IMG
CH = 8


def _nms_kernel(r_ref, c_ref, keep_ref, adj_ref):
    x1 = r_ref[0:1, :]
    y1 = r_ref[1:2, :]
    x2 = r_ref[2:3, :]
    y2 = r_ref[3:4, :]
    area = r_ref[4:5, :]
    validv = r_ref[5:6, :]

    def build(ci, _):
        base = ci * CH
        cx1 = c_ref[pl.ds(base, CH), 0:1]
        cy1 = c_ref[pl.ds(base, CH), 1:2]
        cx2 = c_ref[pl.ds(base, CH), 2:3]
        cy2 = c_ref[pl.ds(base, CH), 3:4]
        carea = c_ref[pl.ds(base, CH), 4:5]
        ix1 = jnp.maximum(cx1, x1)
        iy1 = jnp.maximum(cy1, y1)
        ix2 = jnp.minimum(cx2, x2)
        iy2 = jnp.minimum(cy2, y2)
        iw = jnp.maximum(ix2 - ix1, 0.0)
        ih = jnp.maximum(iy2 - iy1, 0.0)
        inter = iw * ih
        union = carea + area - inter
        iou = inter / jnp.maximum(union, 1e-9)
        adj_ref[pl.ds(base, CH), :] = (iou > NMS_THRESH).astype(jnp.float32)
        return 0

    jax.lax.fori_loop(0, KP // CH, build, 0, unroll=True)

    idx = jax.lax.broadcasted_iota(jnp.int32, (1, KP), 1)

    def body(i, carry):
        sup, keep = carry
        row = adj_ref[pl.ds(i, 1), :]
        m = idx == i
        sup_i = jnp.sum(jnp.where(m, sup, 0.0))
        val_i = jnp.sum(jnp.where(m, validv, 0.0))
        ok = val_i * (1.0 - sup_i)
        keep = jnp.where(m, ok, keep)
        sup = jnp.maximum(sup, row * ok)
        return sup, keep

    sup0 = jnp.zeros((1, KP), jnp.float32)
    keep0 = jnp.zeros((1, KP), jnp.float32)
    _, keep = jax.lax.fori_loop(0, KP, body, (sup0, keep0))
    keep_ref[...] = keep


def kernel(boxes, scores, labels):
    masked = jnp.where(scores > SCORE_THRESH, scores, NEG)
    ts, idx = jax.lax.top_k(masked, K)
    tb = jnp.take(boxes, idx, axis=0)
    tl = jnp.take(labels, idx, axis=0)
    valid = ts > SCORE_THRESH

    off = tl.astype(tb.dtype) * CLASS_OFFSET
    cx = tb[:, 0] + off
    cy = tb[:, 1] + off
    w = tb[:, 2]
    h = tb[:, 3]
    ang = tb[:, 4]
    c = jnp.abs(jnp.cos(ang))
    s = jnp.abs(jnp.sin(ang))
    bw = w * c + h * s
    bh = w * s + h * c
    x1 = cx - 0.5 * bw
    y1 = cy - 0.5 * bh
    x2 = cx + 0.5 * bw
    y2 = cy + 0.5 * bh
    area = (x2 - x1) * (y2 - y1)

    pad = lambda v: jnp.pad(v, (0, KP - K))
    zeros = jnp.zeros((KP,), jnp.float32)
    r = jnp.stack([pad(x1), pad(y1), pad(x2), pad(y2), pad(area),
                   pad(valid.astype(jnp.float32)), zeros, zeros])
    cstk = jnp.stack([pad(x1), pad(y1), pad(x2), pad(y2), pad(area)], axis=1)
    cstk = jnp.pad(cstk, ((0, 0), (0, 128 - 5)))

    keep = pl.pallas_call(
        _nms_kernel,
        out_shape=jax.ShapeDtypeStruct((1, KP), jnp.float32),
        scratch_shapes=[pltpu.VMEM((KP, KP), jnp.float32)],
    )(r, cstk)

    keepb = keep[0, :K] > 0.5
    kept_scores = jnp.where(keepb, ts, NEG)
    out_scores, oidx = jax.lax.top_k(kept_scores, M)
    out_boxes = jnp.take(tb, oidx, axis=0)
    out_labels = jnp.take(tl, oidx, axis=0)
    finite = out_scores > 0.5 * NEG
    out_scores = jnp.where(finite, out_scores, 0.0)
    out_boxes = jnp.where(finite[:, None], out_boxes, 0.0)
    out_labels = jnp.where(finite, out_labels, -1)
    return out_boxes, out_labels, out_scores
